# baseline (speedup 1.0000x reference)
import functools

import jax
import jax.numpy as jnp
from jax import lax
from jax.experimental import pallas as pl
from jax.experimental.pallas import tpu as pltpu
from jax.experimental.pallas import tpu_sc as plsc

N = 320000
D = 128
B = 1024
NCORES = 2
NSUB = 16
NW = NCORES * NSUB
SEG_PER_W = B // NW
R = 448
NLANE = 16
DV = D // NLANE
MPAD = R + 2 * NLANE
MB = R + NLANE

_mesh = plsc.VectorSubcoreMesh(core_axis_name="c", subcore_axis_name="s")


@functools.partial(
    pl.kernel,
    mesh=_mesh,
    out_type=(
        jax.ShapeDtypeStruct((B * D,), jnp.float32),
        jax.ShapeDtypeStruct((B * D,), jnp.float32),
    ),
    scratch_types=[
        pltpu.VMEM((2 * R * D,), jnp.float32),
        pltpu.VMEM((2 * MB,), jnp.int32),
        pltpu.VMEM(((SEG_PER_W + 1) * D,), jnp.float32),
        pltpu.VMEM(((SEG_PER_W + 1) * D,), jnp.float32),
        pltpu.VMEM((48,), jnp.int32),
        pltpu.VMEM((D,), jnp.float32),
        pltpu.VMEM((D,), jnp.float32),
        pltpu.SMEM((8,), jnp.int32),
        pltpu.SemaphoreType.DMA,
        pltpu.SemaphoreType.DMA,
    ],
)
def _sc_segreduce(x_hbm, m_hbm, bounds_hbm, sum_hbm, max_hbm,
                  xbuf, mbuf, slab_sum, slab_max, bbuf, acc_sum, acc_max,
                  cur_smem, semx, semm):
    wid = lax.axis_index("c") * NSUB + lax.axis_index("s")
    s0 = wid * SEG_PER_W

    pltpu.sync_copy(bounds_hbm, bbuf)
    bvec = bbuf[pl.ds(wid, NLANE)]
    lo = bvec[0]
    hi = bvec[1]
    base0 = (lo // 8) * 8

    zero16 = jnp.zeros((NLANE,), jnp.float32)
    ninf16 = jnp.full((NLANE,), -jnp.inf, jnp.float32)

    for j in range(SEG_PER_W + 1):
        for k in range(DV):
            slab_sum[pl.ds(j * D + k * NLANE, NLANE)] = zero16
            slab_max[pl.ds(j * D + k * NLANE, NLANE)] = ninf16
    cur_smem[0] = jnp.int32(0)

    ntiles = (hi - base0 + (R - 1)) // R

    def _copies(t):
        base = base0 + t * R
        base_c = jnp.minimum(base, N - R)
        par = t & 1
        xsrc = x_hbm.at[pl.ds(base_c * D, R * D)]
        xdst = xbuf.at[pl.ds(par * (R * D), R * D)]
        msrc = m_hbm.at[pl.ds(base, MB)]
        mdst = mbuf.at[pl.ds(par * MB, MB)]
        return (xsrc, xdst), (msrc, mdst), base_c, par

    def _start(t):
        (xsrc, xdst), (msrc, mdst), _, _ = _copies(t)
        pltpu.async_copy(xsrc, xdst, semx)
        pltpu.async_copy(msrc, mdst, semm)

    @pl.when(ntiles > 0)
    def _prime():
        _start(0)

    def tile_body(t, carry):
        @pl.when(t + 1 < ntiles)
        def _next():
            _start(t + 1)

        (xsrc, xdst), (msrc, mdst), base_c, par = _copies(t)
        pltpu.make_async_copy(xsrc, xdst, semx).wait()
        pltpu.make_async_copy(msrc, mdst, semm).wait()

        base = base0 + t * R
        xoff = par * (R * D) - base_c * D
        moff = par * MB - base

        def group_body(g, accs):
            g_start = base + g * NLANE
            mv = mbuf[pl.ds(moff + g_start, NLANE)]
            cur = cur_smem[0]
            uniform = (mv[NLANE - 1] == mv[0]) & (mv[0] - s0 == cur)

            @pl.when(jnp.logical_not(uniform))
            def slow():
                for k in range(DV):
                    co = cur * D + k * NLANE
                    slab_sum[pl.ds(co, NLANE)] = \
                        slab_sum[pl.ds(co, NLANE)] + accs[k]
                    slab_max[pl.ds(co, NLANE)] = jnp.maximum(
                        slab_max[pl.ds(co, NLANE)], accs[DV + k])
                m_last = jnp.int32(-1)
                for r in range(NLANE):
                    m_r = mv[r]
                    rel_r = m_r - s0
                    ok = (rel_r >= 0) & (rel_r < SEG_PER_W)
                    tgt = jnp.where(ok, rel_r, SEG_PER_W)
                    m_last = jnp.where(ok, m_r, m_last)
                    xrow = jnp.minimum(g_start + r, N - 1)
                    xs = [xbuf[pl.ds(xoff + xrow * D + k * NLANE, NLANE)]
                          for k in range(DV)]
                    for k in range(DV):
                        ro = tgt * D + k * NLANE
                        slab_sum[pl.ds(ro, NLANE)] = \
                            slab_sum[pl.ds(ro, NLANE)] + xs[k]
                        slab_max[pl.ds(ro, NLANE)] = \
                            jnp.maximum(slab_max[pl.ds(ro, NLANE)], xs[k])
                cur_smem[0] = jnp.where(m_last >= 0, m_last - s0, cur)

            sums = list(accs[:DV])
            maxs = list(accs[DV:])
            for r in range(NLANE):
                xs = [xbuf[pl.ds(xoff + (g_start + r) * D + k * NLANE,
                                 NLANE)] for k in range(DV)]
                sums = [sums[k] + xs[k] for k in range(DV)]
                maxs = [jnp.maximum(maxs[k], xs[k]) for k in range(DV)]
            new_sums = tuple(jnp.where(uniform, sums[k], zero16)
                             for k in range(DV))
            new_maxs = tuple(jnp.where(uniform, maxs[k], ninf16)
                             for k in range(DV))
            return new_sums + new_maxs

        return lax.fori_loop(0, R // NLANE, group_body, carry)

    acc0 = (zero16,) * DV + (ninf16,) * DV
    faccs = lax.fori_loop(0, ntiles, tile_body, acc0)

    cur_rel = cur_smem[0]
    for k in range(DV):
        co = cur_rel * D + k * NLANE
        slab_sum[pl.ds(co, NLANE)] = \
            slab_sum[pl.ds(co, NLANE)] + faccs[k]
        slab_max[pl.ds(co, NLANE)] = jnp.maximum(
            slab_max[pl.ds(co, NLANE)], faccs[DV + k])

    pltpu.sync_copy(slab_sum.at[pl.ds(0, SEG_PER_W * D)],
                    sum_hbm.at[pl.ds(s0 * D, SEG_PER_W * D)])
    pltpu.sync_copy(slab_max.at[pl.ds(0, SEG_PER_W * D)],
                    max_hbm.at[pl.ds(s0 * D, SEG_PER_W * D)])


def _tc_merge_body(sum_ref, max_ref, wmax_ref, wsum_ref, b_ref, out_ref):
    mx = max_ref[...]
    mx = jnp.where(jnp.isfinite(mx), mx, 0.0)
    acc = lax.dot_general(mx, wmax_ref[...], (((1,), (1,)), ((), ())),
                          preferred_element_type=jnp.float32)
    acc += lax.dot_general(sum_ref[...], wsum_ref[...], (((1,), (1,)), ((), ())),
                           preferred_element_type=jnp.float32)
    out_ref[...] = acc + b_ref[...]


_tc_merge = pl.pallas_call(
    _tc_merge_body,
    out_shape=jax.ShapeDtypeStruct((B, D), jnp.float32),
)


def kernel(x, membership, merge_W, merge_b):
    edges = jnp.arange(NW + 1, dtype=jnp.int32) * SEG_PER_W
    bounds = jnp.searchsorted(membership, edges, side="left",
                              method="compare_all").astype(jnp.int32)
    bounds = jnp.zeros((48,), jnp.int32).at[:NW + 1].set(bounds)

    m_pad = jnp.concatenate(
        [membership, jnp.full((MPAD,), B, jnp.int32)], axis=0)

    seg_sum, seg_max = _sc_segreduce(x.reshape(-1), m_pad, bounds)
    seg_sum = seg_sum.reshape(B, D)
    seg_max = seg_max.reshape(B, D)
    return _tc_merge(seg_sum, seg_max, merge_W[:, :D], merge_W[:, D:],
                     merge_b.reshape(1, D))


# speedup vs baseline: 1.0038x; 1.0038x over previous
import functools

import jax
import jax.numpy as jnp
from jax import lax
from jax.experimental import pallas as pl
from jax.experimental.pallas import tpu as pltpu
from jax.experimental.pallas import tpu_sc as plsc

N = 320000
D = 128
B = 1024
NCORES = 2
NSUB = 16
NW = NCORES * NSUB
SEG_PER_W = B // NW
R = 384
NLANE = 16
DV = D // NLANE
MPAD = R + 2 * NLANE
MB = R + NLANE

_mesh = plsc.VectorSubcoreMesh(core_axis_name="c", subcore_axis_name="s")


@functools.partial(
    pl.kernel,
    mesh=_mesh,
    out_type=(
        jax.ShapeDtypeStruct((B * D,), jnp.float32),
        jax.ShapeDtypeStruct((B * D,), jnp.float32),
    ),
    scratch_types=[
        pltpu.VMEM((2 * R * D,), jnp.float32),
        pltpu.VMEM((2 * MB,), jnp.int32),
        pltpu.VMEM(((SEG_PER_W + 1) * D,), jnp.float32),
        pltpu.VMEM(((SEG_PER_W + 1) * D,), jnp.float32),
        pltpu.VMEM((48,), jnp.int32),
        pltpu.SMEM((8,), jnp.int32),
        pltpu.SemaphoreType.DMA,
        pltpu.SemaphoreType.DMA,
    ],
)
def _sc_segreduce(x_hbm, m_hbm, bounds_hbm, sum_hbm, max_hbm,
                  xbuf, mbuf, slab_sum, slab_max, bbuf, cur_smem, semx, semm):
    wid = lax.axis_index("c") * NSUB + lax.axis_index("s")
    s0 = wid * SEG_PER_W

    pltpu.sync_copy(bounds_hbm, bbuf)
    bvec = bbuf[pl.ds(wid, NLANE)]
    lo = bvec[0]
    hi = bvec[1]
    base0 = (lo // 8) * 8

    zero16 = jnp.zeros((NLANE,), jnp.float32)
    ninf16 = jnp.full((NLANE,), -jnp.inf, jnp.float32)

    for j in range(SEG_PER_W + 1):
        for k in range(DV):
            slab_sum[pl.ds(j * D + k * NLANE, NLANE)] = zero16
            slab_max[pl.ds(j * D + k * NLANE, NLANE)] = ninf16
    cur_smem[0] = jnp.int32(0)

    ntiles = (hi - base0 + (R - 1)) // R

    def _copies(t):
        base = base0 + t * R
        base_c = jnp.minimum(base, N - R)
        par = t & 1
        xsrc = x_hbm.at[pl.ds(base_c * D, R * D)]
        xdst = xbuf.at[pl.ds(par * (R * D), R * D)]
        msrc = m_hbm.at[pl.ds(base, MB)]
        mdst = mbuf.at[pl.ds(par * MB, MB)]
        return (xsrc, xdst), (msrc, mdst), base_c, par

    def _start(t):
        (xsrc, xdst), (msrc, mdst), _, _ = _copies(t)
        pltpu.async_copy(xsrc, xdst, semx)
        pltpu.async_copy(msrc, mdst, semm)

    @pl.when(ntiles > 0)
    def _prime():
        _start(0)

    def tile_body(t, carry):
        @pl.when(t + 1 < ntiles)
        def _next():
            _start(t + 1)

        (xsrc, xdst), (msrc, mdst), base_c, par = _copies(t)
        pltpu.make_async_copy(xsrc, xdst, semx).wait()
        pltpu.make_async_copy(msrc, mdst, semm).wait()

        base = base0 + t * R
        xoff = par * (R * D) - base_c * D
        moff = par * MB - base

        def group_body(g, accs):
            g_start = base + g * NLANE
            mv = mbuf[pl.ds(moff + g_start, NLANE)]
            cur = cur_smem[0]
            uniform = (mv[NLANE - 1] == mv[0]) & (mv[0] - s0 == cur)

            @pl.when(jnp.logical_not(uniform))
            def slow():
                for k in range(DV):
                    co = cur * D + k * NLANE
                    slab_sum[pl.ds(co, NLANE)] = \
                        slab_sum[pl.ds(co, NLANE)] + accs[k]
                    slab_max[pl.ds(co, NLANE)] = jnp.maximum(
                        slab_max[pl.ds(co, NLANE)], accs[DV + k])
                m_last = jnp.int32(-1)
                for r in range(NLANE):
                    m_r = mv[r]
                    rel_r = m_r - s0
                    ok = (rel_r >= 0) & (rel_r < SEG_PER_W)
                    tgt = jnp.where(ok, rel_r, SEG_PER_W)
                    m_last = jnp.where(ok, m_r, m_last)
                    xrow = jnp.minimum(g_start + r, N - 1)
                    xs = [xbuf[pl.ds(xoff + xrow * D + k * NLANE, NLANE)]
                          for k in range(DV)]
                    for k in range(DV):
                        ro = tgt * D + k * NLANE
                        slab_sum[pl.ds(ro, NLANE)] = \
                            slab_sum[pl.ds(ro, NLANE)] + xs[k]
                        slab_max[pl.ds(ro, NLANE)] = \
                            jnp.maximum(slab_max[pl.ds(ro, NLANE)], xs[k])
                cur_smem[0] = jnp.where(m_last >= 0, m_last - s0, cur)

            sums = list(accs[:DV])
            maxs = list(accs[DV:])
            for r in range(NLANE):
                xs = [xbuf[pl.ds(xoff + (g_start + r) * D + k * NLANE,
                                 NLANE)] for k in range(DV)]
                sums = [sums[k] + xs[k] for k in range(DV)]
                maxs = [jnp.maximum(maxs[k], xs[k]) for k in range(DV)]
            new_sums = tuple(jnp.where(uniform, sums[k], zero16)
                             for k in range(DV))
            new_maxs = tuple(jnp.where(uniform, maxs[k], ninf16)
                             for k in range(DV))
            return new_sums + new_maxs

        return lax.fori_loop(0, R // NLANE, group_body, carry)

    acc0 = (zero16,) * DV + (ninf16,) * DV
    faccs = lax.fori_loop(0, ntiles, tile_body, acc0)

    cur_rel = cur_smem[0]
    for k in range(DV):
        co = cur_rel * D + k * NLANE
        slab_sum[pl.ds(co, NLANE)] = \
            slab_sum[pl.ds(co, NLANE)] + faccs[k]
        slab_max[pl.ds(co, NLANE)] = jnp.maximum(
            slab_max[pl.ds(co, NLANE)], faccs[DV + k])

    pltpu.sync_copy(slab_sum.at[pl.ds(0, SEG_PER_W * D)],
                    sum_hbm.at[pl.ds(s0 * D, SEG_PER_W * D)])
    pltpu.sync_copy(slab_max.at[pl.ds(0, SEG_PER_W * D)],
                    max_hbm.at[pl.ds(s0 * D, SEG_PER_W * D)])


def _tc_merge_body(sum_ref, max_ref, wmax_ref, wsum_ref, b_ref, out_ref):
    mx = max_ref[...]
    mx = jnp.where(jnp.isfinite(mx), mx, 0.0)
    acc = lax.dot_general(mx, wmax_ref[...], (((1,), (1,)), ((), ())),
                          preferred_element_type=jnp.float32)
    acc += lax.dot_general(sum_ref[...], wsum_ref[...], (((1,), (1,)), ((), ())),
                           preferred_element_type=jnp.float32)
    out_ref[...] = acc + b_ref[...]


_tc_merge = pl.pallas_call(
    _tc_merge_body,
    out_shape=jax.ShapeDtypeStruct((B, D), jnp.float32),
)


def kernel(x, membership, merge_W, merge_b):
    edges = jnp.arange(NW + 1, dtype=jnp.int32) * SEG_PER_W
    bounds = jnp.searchsorted(membership, edges, side="left",
                              method="compare_all").astype(jnp.int32)
    bounds = jnp.zeros((48,), jnp.int32).at[:NW + 1].set(bounds)

    m_pad = jnp.concatenate(
        [membership, jnp.full((MPAD,), B, jnp.int32)], axis=0)

    seg_sum, seg_max = _sc_segreduce(x.reshape(-1), m_pad, bounds)
    seg_sum = seg_sum.reshape(B, D)
    seg_max = seg_max.reshape(B, D)
    return _tc_merge(seg_sum, seg_max, merge_W[:, :D], merge_W[:, D:],
                     merge_b.reshape(1, D))
